# baseline (speedup 1.0000x reference)
import functools

import jax
import jax.numpy as jnp
from jax import lax
from jax.experimental import pallas as pl
from jax.experimental.pallas import tpu as pltpu
from jax.experimental.pallas import tpu_sc as plsc

_NC, _NS, _L = 2, 16, 16
_NW = _NC * _NS


def _make_sc_add(n_elems, pos_elems, chunk_elems):
    n_chunks = n_elems // (_NW * chunk_elems)
    mesh = plsc.VectorSubcoreMesh(core_axis_name="c", subcore_axis_name="s")

    @functools.partial(
        pl.kernel,
        out_type=jax.ShapeDtypeStruct((n_elems,), jnp.float32),
        mesh=mesh,
        scratch_types=[
            pltpu.VMEM((chunk_elems,), jnp.float32),
            pltpu.VMEM((chunk_elems,), jnp.float32),
            pltpu.SemaphoreType.DMA,
            pltpu.SemaphoreType.DMA,
        ],
    )
    def sc_add(x_hbm, pos_hbm, out_hbm, xbuf, pbuf, sem_x, sem_p):
        wid = lax.axis_index("s") * _NC + lax.axis_index("c")
        base = wid * (n_chunks * chunk_elems)
        pbase = lax.rem(base, pos_elems)

        def chunk_body(c, carry):
            off = c * chunk_elems
            cp_x = pltpu.make_async_copy(
                x_hbm.at[pl.ds(base + off, chunk_elems)], xbuf, sem_x)
            cp_x.start()
            cp_p = pltpu.make_async_copy(
                pos_hbm.at[pl.ds(pbase + off, chunk_elems)], pbuf, sem_p)
            cp_p.start()
            cp_x.wait()
            cp_p.wait()

            def add_body(j, carry2):
                sl = pl.ds(j * _L, _L)
                xbuf[sl] = xbuf[sl] + pbuf[sl]
                return carry2

            lax.fori_loop(0, chunk_elems // _L, add_body, 0, unroll=8)

            pltpu.sync_copy(xbuf, out_hbm.at[pl.ds(base + off, chunk_elems)])
            return carry

        lax.fori_loop(0, n_chunks, chunk_body, 0)

    return sc_add


def kernel(x, pos_table):
    B, S, D = x.shape
    n_elems = B * S * D
    pos_elems = S * D
    chunk_elems = 32 * D
    out = _make_sc_add(n_elems, pos_elems, chunk_elems)(
        x.reshape(n_elems), pos_table.reshape(pos_elems))
    return out.reshape(B, S, D)


# speedup vs baseline: 1.1088x; 1.1088x over previous
import functools

import jax
import jax.numpy as jnp
from jax import lax
from jax.experimental import pallas as pl
from jax.experimental.pallas import tpu as pltpu
from jax.experimental.pallas import tpu_sc as plsc

_NC, _NS, _L = 2, 16, 16
_NW = _NC * _NS


def _make_sc_add(B, S, D):
    CH = 16
    CHE = CH * D
    SPW = S // _NW
    NCH = SPW // CH
    G = NCH * B
    assert S % _NW == 0 and SPW % CH == 0 and (CH * D) % _L == 0

    mesh = plsc.VectorSubcoreMesh(core_axis_name="c", subcore_axis_name="s")

    @functools.partial(
        pl.kernel,
        out_type=jax.ShapeDtypeStruct((B * S * D,), jnp.float32),
        mesh=mesh,
        scratch_types=[
            pltpu.VMEM((CHE,), jnp.float32),
            pltpu.VMEM((CHE,), jnp.float32),
            pltpu.VMEM((CHE,), jnp.float32),
            pltpu.VMEM((CHE,), jnp.float32),
            pltpu.VMEM((CHE,), jnp.float32),
            pltpu.SemaphoreType.DMA,
            pltpu.SemaphoreType.DMA,
            pltpu.SemaphoreType.DMA,
            pltpu.SemaphoreType.DMA,
        ],
    )
    def sc_add(x_hbm, pos_hbm, out_hbm, xb0, xb1, ob0, ob1, pbuf,
               sx0, sx1, so0, so1):
        w = lax.axis_index("s") * _NC + lax.axis_index("c")
        s_base = w * (SPW * D)
        xbufs, obufs = (xb0, xb1), (ob0, ob1)
        sxs, sos = (sx0, sx1), (so0, so1)

        def x_off(g):
            return lax.rem(g, B) * (S * D) + s_base + lax.div(g, B) * CHE

        for j in range(2):
            pltpu.make_async_copy(
                x_hbm.at[pl.ds(x_off(jnp.int32(j)), CHE)], xbufs[j],
                sxs[j]).start()

        def body(i, carry):
            for j in range(2):
                g = 2 * i + j
                xb, ob, sx, so = xbufs[j], obufs[j], sxs[j], sos[j]

                @pl.when(lax.rem(g, B) == 0)
                def _():
                    pltpu.sync_copy(
                        pos_hbm.at[pl.ds(s_base + lax.div(g, B) * CHE, CHE)],
                        pbuf)

                pltpu.make_async_copy(
                    x_hbm.at[pl.ds(x_off(g), CHE)], xb, sx).wait()

                @pl.when(g >= 2)
                def _():
                    pltpu.make_async_copy(
                        ob, out_hbm.at[pl.ds(x_off(g - 2), CHE)], so).wait()

                def add_body(k, carry2):
                    sl = pl.ds(k * _L, _L)
                    ob[sl] = xb[sl] + pbuf[sl]
                    return carry2

                lax.fori_loop(0, CHE // _L, add_body, 0, unroll=8)

                @pl.when(g + 2 < G)
                def _():
                    pltpu.make_async_copy(
                        x_hbm.at[pl.ds(x_off(g + 2), CHE)], xb, sx).start()

                pltpu.make_async_copy(
                    ob, out_hbm.at[pl.ds(x_off(g), CHE)], so).start()
            return carry

        lax.fori_loop(0, G // 2, body, 0)

        for j in range(2):
            pltpu.make_async_copy(
                obufs[j], out_hbm.at[pl.ds(x_off(jnp.int32(G - 2 + j)), CHE)],
                sos[j]).wait()

    return sc_add


def kernel(x, pos_table):
    B, S, D = x.shape
    out = _make_sc_add(B, S, D)(
        x.reshape(B * S * D), pos_table.reshape(S * D))
    return out.reshape(B, S, D)


# speedup vs baseline: 1.8260x; 1.6469x over previous
import functools

import jax
import jax.numpy as jnp
from jax import lax
from jax.experimental import pallas as pl
from jax.experimental.pallas import tpu as pltpu
from jax.experimental.pallas import tpu_sc as plsc

_NC, _NS, _L = 2, 16, 16
_NW = _NC * _NS


def _make_sc_add(B, S, D):
    CH = 8
    CHE = CH * D
    SPW = S // _NW
    NCH = SPW // CH
    G = NCH * B
    NB = 4
    UNROLL = 2 * B
    assert S % _NW == 0 and SPW % CH == 0 and G % UNROLL == 0 and B == NB

    mesh = plsc.VectorSubcoreMesh(core_axis_name="c", subcore_axis_name="s")

    @functools.partial(
        pl.kernel,
        out_type=jax.ShapeDtypeStruct((B * S * D,), jnp.float32),
        mesh=mesh,
        scratch_types=(
            [pltpu.VMEM((CHE,), jnp.float32) for _ in range(2 * NB + 2)]
            + [pltpu.SemaphoreType.DMA for _ in range(2 * NB + 2)]
        ),
    )
    def sc_add(x_hbm, pos_hbm, out_hbm, *refs):
        xbufs = refs[0:NB]
        obufs = refs[NB:2 * NB]
        pbufs = refs[2 * NB:2 * NB + 2]
        sxs = refs[2 * NB + 2:3 * NB + 2]
        sos = refs[3 * NB + 2:4 * NB + 2]
        sps = refs[4 * NB + 2:4 * NB + 4]

        w = lax.axis_index("s") * _NC + lax.axis_index("c")
        s_base = w * (SPW * D)

        def x_off(g):
            return lax.rem(g, B) * (S * D) + s_base + lax.div(g, B) * CHE

        def start_x(g, buf, sem):
            pltpu.make_async_copy(
                x_hbm.at[pl.ds(x_off(g), CHE)], buf, sem).start()

        def start_pos(c, buf, sem):
            pltpu.make_async_copy(
                pos_hbm.at[pl.ds(s_base + c * CHE, CHE)], buf, sem).start()

        for j in range(NB):
            start_x(jnp.int32(j), xbufs[j], sxs[j])
        for q in range(2):
            start_pos(jnp.int32(q), pbufs[q], sps[q])

        def body(i, carry):
            for j in range(UNROLL):
                g = UNROLL * i + j
                xb, ob = xbufs[j % NB], obufs[j % NB]
                sx, so = sxs[j % NB], sos[j % NB]
                q = j // B
                pb, sp = pbufs[q], sps[q]
                c = 2 * i + q

                if j % B == 0:
                    pltpu.make_async_copy(
                        pos_hbm.at[pl.ds(s_base + c * CHE, CHE)], pb,
                        sp).wait()

                pltpu.make_async_copy(
                    x_hbm.at[pl.ds(x_off(g), CHE)], xb, sx).wait()

                @pl.when(g >= NB)
                def _():
                    pltpu.make_async_copy(
                        ob, out_hbm.at[pl.ds(x_off(g - NB), CHE)], so).wait()

                U = 8

                def add_body(blk, carry2):
                    k0 = blk * U
                    sls = [pl.ds((k0 + t) * _L, _L) for t in range(U)]
                    xs = [xb[sl] for sl in sls]
                    ps = [pb[sl] for sl in sls]
                    for sl, xv, pv in zip(sls, xs, ps):
                        ob[sl] = xv + pv
                    return carry2

                lax.fori_loop(0, CHE // (_L * U), add_body, 0)

                @pl.when(g + NB < G)
                def _():
                    start_x(g + NB, xb, sx)

                pltpu.make_async_copy(
                    ob, out_hbm.at[pl.ds(x_off(g), CHE)], so).start()

                if j % B == B - 1:
                    @pl.when(c + 2 < NCH)
                    def _():
                        start_pos(c + 2, pb, sp)
            return carry

        lax.fori_loop(0, G // UNROLL, body, 0)

        for j in range(NB):
            pltpu.make_async_copy(
                obufs[j], out_hbm.at[pl.ds(x_off(jnp.int32(G - NB + j)), CHE)],
                sos[j]).wait()

    return sc_add


def kernel(x, pos_table):
    B, S, D = x.shape
    out = _make_sc_add(B, S, D)(
        x.reshape(B * S * D), pos_table.reshape(S * D))
    return out.reshape(B, S, D)


# speedup vs baseline: 4.7773x; 2.6162x over previous
import functools

import jax
import jax.numpy as jnp
from jax import lax
from jax.experimental import pallas as pl
from jax.experimental.pallas import tpu as pltpu
from jax.experimental.pallas import tpu_sc as plsc

_NC, _NS, _L = 2, 16, 16
_NW = _NC * _NS


def _make_sc_add(B, S, D):
    CH = 8
    SPW = S // _NW
    NCH = SPW // CH
    G = NCH * B
    NB = 4
    UNROLL = 2 * B
    assert S % _NW == 0 and SPW % CH == 0 and G % UNROLL == 0 and B == NB
    assert D % _L == 0

    mesh = plsc.VectorSubcoreMesh(core_axis_name="c", subcore_axis_name="s")

    @functools.partial(
        pl.kernel,
        out_type=jax.ShapeDtypeStruct((B * S, D), jnp.float32),
        mesh=mesh,
        compiler_params=pltpu.CompilerParams(use_tc_tiling_on_sc=True),
        scratch_types=(
            [pltpu.VMEM((CH, D), jnp.float32) for _ in range(2 * NB + 2)]
            + [pltpu.SemaphoreType.DMA for _ in range(2 * NB + 2)]
        ),
    )
    def sc_add(x_hbm, pos_hbm, out_hbm, *refs):
        xbufs = refs[0:NB]
        obufs = refs[NB:2 * NB]
        pbufs = refs[2 * NB:2 * NB + 2]
        sxs = refs[2 * NB + 2:3 * NB + 2]
        sos = refs[3 * NB + 2:4 * NB + 2]
        sps = refs[4 * NB + 2:4 * NB + 4]

        w = lax.axis_index("s") * _NC + lax.axis_index("c")
        s_row = w * SPW

        def row_off(g):
            return lax.rem(g, B) * S + s_row + lax.div(g, B) * CH

        def start_x(g, buf, sem):
            pltpu.make_async_copy(
                x_hbm.at[pl.ds(row_off(g), CH)], buf, sem).start()

        def start_pos(c, buf, sem):
            pltpu.make_async_copy(
                pos_hbm.at[pl.ds(s_row + c * CH, CH)], buf, sem).start()

        for j in range(NB):
            start_x(jnp.int32(j), xbufs[j], sxs[j])
        for q in range(2):
            start_pos(jnp.int32(q), pbufs[q], sps[q])

        def body(i, carry):
            for j in range(UNROLL):
                g = UNROLL * i + j
                xb, ob = xbufs[j % NB], obufs[j % NB]
                sx, so = sxs[j % NB], sos[j % NB]
                q = j // B
                pb, sp = pbufs[q], sps[q]
                c = 2 * i + q

                if j % B == 0:
                    pltpu.make_async_copy(
                        pos_hbm.at[pl.ds(s_row + c * CH, CH)], pb, sp).wait()

                pltpu.make_async_copy(
                    x_hbm.at[pl.ds(row_off(g), CH)], xb, sx).wait()

                @pl.when(g >= NB)
                def _():
                    pltpu.make_async_copy(
                        ob, out_hbm.at[pl.ds(row_off(g - NB), CH)], so).wait()

                def add_body(k, carry2):
                    sl = pl.ds(k * _L, _L)
                    xs = [xb[r, sl] for r in range(CH)]
                    ps = [pb[r, sl] for r in range(CH)]
                    for r in range(CH):
                        ob[r, sl] = xs[r] + ps[r]
                    return carry2

                lax.fori_loop(0, D // _L, add_body, 0)

                @pl.when(g + NB < G)
                def _():
                    start_x(g + NB, xb, sx)

                pltpu.make_async_copy(
                    ob, out_hbm.at[pl.ds(row_off(g), CH)], so).start()

                if j % B == B - 1:
                    @pl.when(c + 2 < NCH)
                    def _():
                        start_pos(c + 2, pb, sp)
            return carry

        lax.fori_loop(0, G // UNROLL, body, 0)

        for j in range(NB):
            pltpu.make_async_copy(
                obufs[j],
                out_hbm.at[pl.ds(row_off(jnp.int32(G - NB + j)), CH)],
                sos[j]).wait()

    return sc_add


def kernel(x, pos_table):
    B, S, D = x.shape
    out = _make_sc_add(B, S, D)(x.reshape(B * S, D), pos_table)
    return out.reshape(B, S, D)


# speedup vs baseline: 5.2061x; 1.0898x over previous
import functools

import jax
import jax.numpy as jnp
from jax import lax
from jax.experimental import pallas as pl
from jax.experimental.pallas import tpu as pltpu
from jax.experimental.pallas import tpu_sc as plsc

_NC, _NS, _L = 2, 16, 16
_NW = _NC * _NS


def _make_sc_add(B, S, D):
    CH = 8
    SPW = S // _NW
    NCH = SPW // CH
    G = NCH * B
    NR = 8
    UNROLL = 2 * B
    assert S % _NW == 0 and SPW % CH == 0 and G % UNROLL == 0
    assert B == 4 and UNROLL == NR and D % _L == 0

    mesh = plsc.VectorSubcoreMesh(core_axis_name="c", subcore_axis_name="s")

    @functools.partial(
        pl.kernel,
        out_type=jax.ShapeDtypeStruct((B * S, D), jnp.float32),
        mesh=mesh,
        compiler_params=pltpu.CompilerParams(use_tc_tiling_on_sc=True),
        scratch_types=(
            [pltpu.VMEM((CH, D), jnp.float32) for _ in range(NR + 2)]
            + [pltpu.SemaphoreType.DMA for _ in range(2 * NR + 2)]
        ),
    )
    def sc_add(x_hbm, pos_hbm, out_hbm, *refs):
        bufs = refs[0:NR]
        pbufs = refs[NR:NR + 2]
        sxs = refs[NR + 2:2 * NR + 2]
        sos = refs[2 * NR + 2:3 * NR + 2]
        sps = refs[3 * NR + 2:3 * NR + 4]

        w = lax.axis_index("s") * _NC + lax.axis_index("c")
        s_row = w * SPW

        def row_off(g):
            return lax.rem(g, B) * S + s_row + lax.div(g, B) * CH

        def start_x(g, buf, sem):
            pltpu.make_async_copy(
                x_hbm.at[pl.ds(row_off(g), CH)], buf, sem).start()

        def start_pos(c, buf, sem):
            pltpu.make_async_copy(
                pos_hbm.at[pl.ds(s_row + c * CH, CH)], buf, sem).start()

        for j in range(NR // 2):
            start_x(jnp.int32(j), bufs[j], sxs[j])
        for q in range(2):
            start_pos(jnp.int32(q), pbufs[q], sps[q])

        def body(i, carry):
            for j in range(UNROLL):
                g = UNROLL * i + j
                xb, sx, so = bufs[j], sxs[j], sos[j]
                q = j // B
                pb, sp = pbufs[q], sps[q]
                c = 2 * i + q

                if j % B == 0:
                    pltpu.make_async_copy(
                        pos_hbm.at[pl.ds(s_row + c * CH, CH)], pb, sp).wait()

                pltpu.make_async_copy(
                    x_hbm.at[pl.ds(row_off(g), CH)], xb, sx).wait()

                U = 4

                def add_body(k, carry2):
                    sl = pl.ds(k * _L, _L)
                    for r0 in range(0, CH, U):
                        ps = [pb[r0 + t, sl] for t in range(U)]
                        for t in range(U):
                            plsc.addupdate(xb.at[r0 + t, sl], ps[t])
                    return carry2

                lax.fori_loop(0, D // _L, add_body, 0)

                pltpu.make_async_copy(
                    xb, out_hbm.at[pl.ds(row_off(g), CH)], so).start()

                j2 = (j + NR // 2) % NR
                g_old, g_new = g - NR // 2, g + NR // 2

                @pl.when(g_old >= 0)
                def _():
                    pltpu.make_async_copy(
                        bufs[j2], out_hbm.at[pl.ds(row_off(g_old), CH)],
                        sos[j2]).wait()

                @pl.when(g_new < G)
                def _():
                    start_x(g_new, bufs[j2], sxs[j2])

                if j % B == B - 1:
                    @pl.when(c + 2 < NCH)
                    def _():
                        start_pos(c + 2, pb, sp)
            return carry

        lax.fori_loop(0, G // UNROLL, body, 0)

        for j in range(NR // 2):
            g_last = G - NR // 2 + j
            pltpu.make_async_copy(
                bufs[g_last % NR],
                out_hbm.at[pl.ds(row_off(jnp.int32(g_last)), CH)],
                sos[g_last % NR]).wait()

    return sc_add


def kernel(x, pos_table):
    B, S, D = x.shape
    out = _make_sc_add(B, S, D)(x.reshape(B * S, D), pos_table)
    return out.reshape(B, S, D)
